# baseline (speedup 1.0000x reference)
import jax
import jax.numpy as jnp
import numpy as np
from jax import lax
from jax.experimental import pallas as pl
from jax.experimental.pallas import tpu as pltpu

_ALPHA = 0.25
_Z = np.int32(0)
_W = 16


def _compat_kernel(h_ref, leftT_ref, rightT_ref, wb_ref, bb_ref, wh_ref,
                   wd_ref, wr1_ref, br1_ref, wr2_ref, br2_ref, r1h_ref,
                   lapT_ref, gold_ref):
    f32 = jnp.float32
    h = h_ref[0]
    n = h.shape[0]

    dn_et = (((1,), (1,)), ((), ()))
    tmp0 = jnp.dot(h, wb_ref[0], preferred_element_type=f32)
    tmp1 = jnp.dot(h, wb_ref[1], preferred_element_type=f32)
    bilin0T = lax.dot_general(h, tmp0, dn_et, preferred_element_type=f32)
    bilin1T = lax.dot_general(h, tmp1, dn_et, preferred_element_type=f32)

    head0 = lax.dot_general(wh_ref[0:1, :], h, dn_et, preferred_element_type=f32)
    head1 = lax.dot_general(wh_ref[1:2, :], h, dn_et, preferred_element_type=f32)
    dep0 = lax.dot_general(h, wd_ref[0:1, :], dn_et, preferred_element_type=f32)
    dep1 = lax.dot_general(h, wd_ref[1:2, :], dn_et, preferred_element_type=f32)

    compat0T = bilin0T + head0 + dep0 + bb_ref[0, 0]
    compat1T = bilin1T + head1 + dep1 + bb_ref[0, 1]

    gold_c = (jnp.sum(compat0T * leftT_ref[0], axis=(0, 1), keepdims=True)
              + jnp.sum(compat1T * rightT_ref[0], axis=(0, 1), keepdims=True))

    aT = jnp.exp(compat0T) + jnp.exp(compat1T)

    z = jnp.dot(h, wr1_ref[...], preferred_element_type=f32) + br1_ref[0:1, :]
    z = 0.5 * z * (1.0 + lax.erf(z * f32(0.7071067811865476)))
    root_col = jnp.dot(z, wr2_ref[...], preferred_element_type=f32) + br2_ref[0, 0]
    gold_r = jnp.sum(root_col * r1h_ref[0], axis=(0, 1), keepdims=True)

    deg = jnp.sum(aT, axis=1, keepdims=True)
    jjd = lax.broadcasted_iota(jnp.int32, (n, n), 0)
    iid = lax.broadcasted_iota(jnp.int32, (n, n), 1)
    lapT = jnp.where(jjd == iid, deg - aT, -aT)
    lapT = jnp.where(iid == 0, jnp.exp(root_col), lapT)
    lapT_ref[0] = lapT
    gold_ref[0] = jnp.broadcast_to(gold_c + gold_r, gold_ref.shape[1:])


def _bdot(lhs, rhs, lc, rc):
    return lax.dot_general(lhs, rhs, (((lc,), (rc,)), ((0,), (0,))),
                           preferred_element_type=jnp.float32,
                           precision=lax.Precision.HIGHEST)


def _lu_kernel(lapT_ref, logabs_ref, sign_ref, mt_ref, qp_ref):
    f32 = jnp.float32
    b, n, _ = lapT_ref.shape
    w = _W
    lapT = lapT_ref[...]
    s = jnp.max(jnp.abs(lapT), axis=2, keepdims=True)
    mt_ref[...] = lapT / s
    scale_log = jnp.sum(jnp.log(s), axis=1)

    neg_inf = f32(-jnp.inf)
    il = lax.broadcasted_iota(jnp.int32, (b, 1, n), 2)
    isub = lax.broadcasted_iota(jnp.int32, (b, n, 1), 1)
    js = lax.broadcasted_iota(jnp.int32, (b, w, 1), 1)
    eye_w = jnp.where(
        lax.broadcasted_iota(jnp.int32, (b, w, w), 1)
        == lax.broadcasted_iota(jnp.int32, (b, w, w), 2), f32(1.0), f32(0.0))

    elim = jnp.zeros((b, 1, n), f32)
    signp = jnp.ones((b, 1, 1), f32)
    logabs = jnp.zeros((b, 1, 1), f32)
    pc = jnp.zeros((b, 1, n), f32)

    for q in range(0, n, w):
        trail = n - q - w

        def body(_, carry, q=q):
            k, elim, signp, logabs, pc = carry
            g = q + k
            colk = mt_ref[:, pl.ds(g, 1), :]
            am = jnp.where(elim > 0.5, neg_inf, jnp.abs(colk))
            m = jnp.max(am, axis=2, keepdims=True)
            piv = jnp.min(jnp.where(am == m, il, jnp.int32(n)),
                          axis=2, keepdims=True)
            is_p = il == piv
            pivot = jnp.sum(jnp.where(is_p, colk, f32(0.0)), axis=2, keepdims=True)
            safe_pivot = jnp.where(pivot == 0.0, f32(1.0), pivot)
            rinv = 1.0 / safe_pivot
            elim = elim + jnp.where(is_p, f32(1.0), f32(0.0))
            f = jnp.where(elim > 0.5, f32(0.0), colk) * rinv

            ptv = mt_ref[:, q:q + w, :]
            u = jnp.sum(jnp.where(is_p, ptv, f32(0.0)), axis=2, keepdims=True)
            mt_ref[:, q:q + w, :] = jnp.where(
                js == k, f, ptv - jnp.where(js > k, u * f, f32(0.0)))
            qp_ref[:, pl.ds(k, 1), :] = jnp.where(is_p, f32(1.0), f32(0.0))

            pc = jnp.where(il == g, piv.astype(f32), pc)
            signp = signp * jnp.sign(pivot)
            logabs = logabs + jnp.log(jnp.abs(pivot))
            return jnp.int32(k + 1), elim, signp, logabs, pc

        init = (jnp.int32(0), elim, signp, logabs, pc)
        _, elim, signp, logabs, pc = lax.fori_loop(0, w, body, init)

        if trail > 0:
            fpan = mt_ref[:, q:q + w, :]
            qp = qp_ref[...]
            a12t = _bdot(mt_ref[:, q + w:, :], qp, 2, 2)
            nt = _bdot(fpan, qp, 2, 2)
            x = eye_w - nt
            at = eye_w + nt
            for _ in range(4):
                ax = _bdot(at, x, 2, 1)
                x = _bdot(x, 2.0 * eye_w - ax, 2, 1)
            u12t = _bdot(a12t, x, 2, 1)
            schur_t = _bdot(u12t, fpan, 2, 1)
            mt_ref[:, q + w:, :] = mt_ref[:, q + w:, :] - schur_t

    pr = lax.transpose(pc, (0, 2, 1))
    inv_cnt = jnp.sum(
        jnp.where(jnp.logical_and(isub < il, pr > pc), f32(1.0), f32(0.0)),
        axis=(1, 2), keepdims=True)
    inv_mod2 = inv_cnt - 2.0 * jnp.floor(inv_cnt * 0.5)
    parity = jnp.where(inv_mod2 > 0.5, f32(-1.0), f32(1.0))
    sign = signp * parity

    logabs_ref[...] = jnp.broadcast_to(logabs[:, 0, :] + scale_log,
                                       logabs_ref.shape)
    sign_ref[...] = jnp.broadcast_to(sign[:, 0, :], sign_ref.shape)


def kernel(h_cat, left_adj, right_adj, W_bilin, b_bilin, W_head, W_dep,
           W_r1, b_r1, W_r2, b_r2, roots):
    f32 = jnp.float32
    b, n, h = h_cat.shape
    roots1h = jax.nn.one_hot(roots, n, dtype=f32).reshape(b, n, 1)

    lapT, gold = pl.pallas_call(
        _compat_kernel,
        grid=(b,),
        in_specs=[
            pl.BlockSpec((1, n, h), lambda i: (i, _Z, _Z)),
            pl.BlockSpec((1, n, n), lambda i: (i, _Z, _Z)),
            pl.BlockSpec((1, n, n), lambda i: (i, _Z, _Z)),
            pl.BlockSpec((2, h, h), lambda i: (_Z, _Z, _Z)),
            pl.BlockSpec((1, 2), lambda i: (_Z, _Z)),
            pl.BlockSpec((2, h), lambda i: (_Z, _Z)),
            pl.BlockSpec((2, h), lambda i: (_Z, _Z)),
            pl.BlockSpec((h, h), lambda i: (_Z, _Z)),
            pl.BlockSpec((1, h), lambda i: (_Z, _Z)),
            pl.BlockSpec((h, 1), lambda i: (_Z, _Z)),
            pl.BlockSpec((1, 1), lambda i: (_Z, _Z)),
            pl.BlockSpec((1, n, 1), lambda i: (i, _Z, _Z)),
        ],
        out_specs=[
            pl.BlockSpec((1, n, n), lambda i: (i, _Z, _Z)),
            pl.BlockSpec((1, 1, 128), lambda i: (i, _Z, _Z)),
        ],
        out_shape=[
            jax.ShapeDtypeStruct((b, n, n), f32),
            jax.ShapeDtypeStruct((b, 1, 128), f32),
        ],
    )(h_cat, jnp.swapaxes(left_adj, 1, 2), jnp.swapaxes(right_adj, 1, 2),
      W_bilin, b_bilin.reshape(1, 2).astype(f32), W_head, W_dep,
      W_r1, b_r1.reshape(1, h).astype(f32), W_r2,
      b_r2.reshape(1, 1).astype(f32), roots1h)

    logabs, sign = pl.pallas_call(
        _lu_kernel,
        out_shape=[
            jax.ShapeDtypeStruct((b, 128), f32),
            jax.ShapeDtypeStruct((b, 128), f32),
        ],
        scratch_shapes=[pltpu.VMEM((b, n, n), f32),
                        pltpu.VMEM((b, _W, n), f32)],
    )(lapT)

    gold_v = gold[:, 0, 0].astype(jnp.float64)
    la = logabs[:, 0].astype(jnp.float64)
    sg = sign[:, 0]
    logdet = jnp.where(sg > 0, la, jnp.nan)
    valid = jnp.logical_and(~jnp.isnan(gold_v), ~jnp.isnan(logdet)).astype(jnp.float64)
    mask = (gold_v <= logdet * valid).astype(jnp.float64)
    loss = (logdet - gold_v) * mask
    loss = jnp.where(jnp.isnan(loss), 0.0, loss)
    return _ALPHA * jnp.sum(loss) / b


# speedup vs baseline: 1.1551x; 1.1551x over previous
import jax
import jax.numpy as jnp
import numpy as np
from jax import lax
from jax.experimental import pallas as pl
from jax.experimental.pallas import tpu as pltpu

_ALPHA = 0.25
_Z = np.int32(0)
_W = 64


def _compat_kernel(h_ref, leftT_ref, rightT_ref, wb_ref, bb_ref, wh_ref,
                   wd_ref, wr1_ref, br1_ref, wr2_ref, br2_ref, r1h_ref,
                   lapT_ref, gold_ref):
    f32 = jnp.float32
    h = h_ref[0]
    n = h.shape[0]

    dn_et = (((1,), (1,)), ((), ()))
    tmp0 = jnp.dot(h, wb_ref[0], preferred_element_type=f32)
    tmp1 = jnp.dot(h, wb_ref[1], preferred_element_type=f32)
    bilin0T = lax.dot_general(h, tmp0, dn_et, preferred_element_type=f32)
    bilin1T = lax.dot_general(h, tmp1, dn_et, preferred_element_type=f32)

    head0 = lax.dot_general(wh_ref[0:1, :], h, dn_et, preferred_element_type=f32)
    head1 = lax.dot_general(wh_ref[1:2, :], h, dn_et, preferred_element_type=f32)
    dep0 = lax.dot_general(h, wd_ref[0:1, :], dn_et, preferred_element_type=f32)
    dep1 = lax.dot_general(h, wd_ref[1:2, :], dn_et, preferred_element_type=f32)

    compat0T = bilin0T + head0 + dep0 + bb_ref[0, 0]
    compat1T = bilin1T + head1 + dep1 + bb_ref[0, 1]

    gold_c = (jnp.sum(compat0T * leftT_ref[0], axis=(0, 1), keepdims=True)
              + jnp.sum(compat1T * rightT_ref[0], axis=(0, 1), keepdims=True))

    aT = jnp.exp(compat0T) + jnp.exp(compat1T)

    z = jnp.dot(h, wr1_ref[...], preferred_element_type=f32) + br1_ref[0:1, :]
    z = 0.5 * z * (1.0 + lax.erf(z * f32(0.7071067811865476)))
    root_col = jnp.dot(z, wr2_ref[...], preferred_element_type=f32) + br2_ref[0, 0]
    gold_r = jnp.sum(root_col * r1h_ref[0], axis=(0, 1), keepdims=True)

    deg = jnp.sum(aT, axis=1, keepdims=True)
    jjd = lax.broadcasted_iota(jnp.int32, (n, n), 0)
    iid = lax.broadcasted_iota(jnp.int32, (n, n), 1)
    lapT = jnp.where(jjd == iid, deg - aT, -aT)
    lapT = jnp.where(iid == 0, jnp.exp(root_col), lapT)
    lapT_ref[0] = lapT
    gold_ref[0] = jnp.broadcast_to(gold_c + gold_r, gold_ref.shape[1:])


def _bdot(lhs, rhs, lc, rc):
    return lax.dot_general(lhs, rhs, (((lc,), (rc,)), ((0,), (0,))),
                           preferred_element_type=jnp.float32,
                           precision=lax.Precision.HIGHEST)


def _lu_kernel(lapT_ref, logabs_ref, sign_ref, mt_ref, qp_ref):
    f32 = jnp.float32
    b, n, _ = lapT_ref.shape
    w = _W
    lapT = lapT_ref[...]
    s = jnp.max(jnp.abs(lapT), axis=2, keepdims=True)
    mt_ref[...] = lapT / s
    scale_log = jnp.sum(jnp.log(s), axis=1)

    neg_inf = f32(-jnp.inf)
    il = lax.broadcasted_iota(jnp.int32, (b, 1, n), 2)
    isub = lax.broadcasted_iota(jnp.int32, (b, n, 1), 1)
    js = lax.broadcasted_iota(jnp.int32, (b, w, 1), 1)
    eye_w = jnp.where(
        lax.broadcasted_iota(jnp.int32, (b, w, w), 1)
        == lax.broadcasted_iota(jnp.int32, (b, w, w), 2), f32(1.0), f32(0.0))

    elim = jnp.zeros((b, 1, n), f32)
    signp = jnp.ones((b, 1, 1), f32)
    logabs = jnp.zeros((b, 1, 1), f32)
    pc = jnp.zeros((b, 1, n), f32)

    for q in range(0, n, w):
        trail = n - q - w

        def body(_, carry, q=q):
            k, elim, signp, logabs, pc = carry
            g = q + k
            colk = mt_ref[:, pl.ds(g, 1), :]
            am = jnp.where(elim > 0.5, neg_inf, jnp.abs(colk))
            m = jnp.max(am, axis=2, keepdims=True)
            piv = jnp.min(jnp.where(am == m, il, jnp.int32(n)),
                          axis=2, keepdims=True)
            is_p = il == piv
            pivot = jnp.sum(jnp.where(is_p, colk, f32(0.0)), axis=2, keepdims=True)
            safe_pivot = jnp.where(pivot == 0.0, f32(1.0), pivot)
            rinv = 1.0 / safe_pivot
            elim = elim + jnp.where(is_p, f32(1.0), f32(0.0))
            f = jnp.where(elim > 0.5, f32(0.0), colk) * rinv

            ptv = mt_ref[:, q:q + w, :]
            u = jnp.sum(jnp.where(is_p, ptv, f32(0.0)), axis=2, keepdims=True)
            mt_ref[:, q:q + w, :] = jnp.where(
                js == k, f, ptv - jnp.where(js > k, u * f, f32(0.0)))
            qp_ref[:, pl.ds(k, 1), :] = jnp.where(is_p, f32(1.0), f32(0.0))

            pc = jnp.where(il == g, piv.astype(f32), pc)
            signp = signp * jnp.sign(pivot)
            logabs = logabs + jnp.log(jnp.abs(pivot))
            return jnp.int32(k + 1), elim, signp, logabs, pc

        init = (jnp.int32(0), elim, signp, logabs, pc)
        _, elim, signp, logabs, pc = lax.fori_loop(0, w, body, init)

        if trail > 0:
            fpan = mt_ref[:, q:q + w, :]
            qp = qp_ref[...]
            a12t = _bdot(mt_ref[:, q + w:, :], qp, 2, 2)
            nt = _bdot(fpan, qp, 2, 2)
            x = eye_w - nt
            at = eye_w + nt
            for _ in range(max(2, (w - 1).bit_length() - 1)):
                ax = _bdot(at, x, 2, 1)
                x = _bdot(x, 2.0 * eye_w - ax, 2, 1)
            u12t = _bdot(a12t, x, 2, 1)
            schur_t = _bdot(u12t, fpan, 2, 1)
            mt_ref[:, q + w:, :] = mt_ref[:, q + w:, :] - schur_t

    pr = lax.transpose(pc, (0, 2, 1))
    inv_cnt = jnp.sum(
        jnp.where(jnp.logical_and(isub < il, pr > pc), f32(1.0), f32(0.0)),
        axis=(1, 2), keepdims=True)
    inv_mod2 = inv_cnt - 2.0 * jnp.floor(inv_cnt * 0.5)
    parity = jnp.where(inv_mod2 > 0.5, f32(-1.0), f32(1.0))
    sign = signp * parity

    logabs_ref[...] = jnp.broadcast_to(logabs[:, 0, :] + scale_log,
                                       logabs_ref.shape)
    sign_ref[...] = jnp.broadcast_to(sign[:, 0, :], sign_ref.shape)


def kernel(h_cat, left_adj, right_adj, W_bilin, b_bilin, W_head, W_dep,
           W_r1, b_r1, W_r2, b_r2, roots):
    f32 = jnp.float32
    b, n, h = h_cat.shape
    roots1h = jax.nn.one_hot(roots, n, dtype=f32).reshape(b, n, 1)

    lapT, gold = pl.pallas_call(
        _compat_kernel,
        grid=(b,),
        in_specs=[
            pl.BlockSpec((1, n, h), lambda i: (i, _Z, _Z)),
            pl.BlockSpec((1, n, n), lambda i: (i, _Z, _Z)),
            pl.BlockSpec((1, n, n), lambda i: (i, _Z, _Z)),
            pl.BlockSpec((2, h, h), lambda i: (_Z, _Z, _Z)),
            pl.BlockSpec((1, 2), lambda i: (_Z, _Z)),
            pl.BlockSpec((2, h), lambda i: (_Z, _Z)),
            pl.BlockSpec((2, h), lambda i: (_Z, _Z)),
            pl.BlockSpec((h, h), lambda i: (_Z, _Z)),
            pl.BlockSpec((1, h), lambda i: (_Z, _Z)),
            pl.BlockSpec((h, 1), lambda i: (_Z, _Z)),
            pl.BlockSpec((1, 1), lambda i: (_Z, _Z)),
            pl.BlockSpec((1, n, 1), lambda i: (i, _Z, _Z)),
        ],
        out_specs=[
            pl.BlockSpec((1, n, n), lambda i: (i, _Z, _Z)),
            pl.BlockSpec((1, 1, 128), lambda i: (i, _Z, _Z)),
        ],
        out_shape=[
            jax.ShapeDtypeStruct((b, n, n), f32),
            jax.ShapeDtypeStruct((b, 1, 128), f32),
        ],
    )(h_cat, jnp.swapaxes(left_adj, 1, 2), jnp.swapaxes(right_adj, 1, 2),
      W_bilin, b_bilin.reshape(1, 2).astype(f32), W_head, W_dep,
      W_r1, b_r1.reshape(1, h).astype(f32), W_r2,
      b_r2.reshape(1, 1).astype(f32), roots1h)

    logabs, sign = pl.pallas_call(
        _lu_kernel,
        out_shape=[
            jax.ShapeDtypeStruct((b, 128), f32),
            jax.ShapeDtypeStruct((b, 128), f32),
        ],
        scratch_shapes=[pltpu.VMEM((b, n, n), f32),
                        pltpu.VMEM((b, _W, n), f32)],
    )(lapT)

    gold_v = gold[:, 0, 0].astype(jnp.float64)
    la = logabs[:, 0].astype(jnp.float64)
    sg = sign[:, 0]
    logdet = jnp.where(sg > 0, la, jnp.nan)
    valid = jnp.logical_and(~jnp.isnan(gold_v), ~jnp.isnan(logdet)).astype(jnp.float64)
    mask = (gold_v <= logdet * valid).astype(jnp.float64)
    loss = (logdet - gold_v) * mask
    loss = jnp.where(jnp.isnan(loss), 0.0, loss)
    return _ALPHA * jnp.sum(loss) / b


# speedup vs baseline: 1.2876x; 1.1147x over previous
import jax
import jax.numpy as jnp
import numpy as np
from jax import lax
from jax.experimental import pallas as pl
from jax.experimental.pallas import tpu as pltpu

_ALPHA = 0.25
_Z = np.int32(0)
_W = 64


def _compat_kernel(h_ref, leftT_ref, rightT_ref, wb_ref, bb_ref, wh_ref,
                   wd_ref, wr1_ref, br1_ref, wr2_ref, br2_ref, r1h_ref,
                   lapT_ref, gold_ref):
    f32 = jnp.float32
    h = h_ref[0]
    n = h.shape[0]

    dn_et = (((1,), (1,)), ((), ()))
    tmp0 = jnp.dot(h, wb_ref[0], preferred_element_type=f32)
    tmp1 = jnp.dot(h, wb_ref[1], preferred_element_type=f32)
    bilin0T = lax.dot_general(h, tmp0, dn_et, preferred_element_type=f32)
    bilin1T = lax.dot_general(h, tmp1, dn_et, preferred_element_type=f32)

    head0 = lax.dot_general(wh_ref[0:1, :], h, dn_et, preferred_element_type=f32)
    head1 = lax.dot_general(wh_ref[1:2, :], h, dn_et, preferred_element_type=f32)
    dep0 = lax.dot_general(h, wd_ref[0:1, :], dn_et, preferred_element_type=f32)
    dep1 = lax.dot_general(h, wd_ref[1:2, :], dn_et, preferred_element_type=f32)

    compat0T = bilin0T + head0 + dep0 + bb_ref[0, 0]
    compat1T = bilin1T + head1 + dep1 + bb_ref[0, 1]

    gold_c = (jnp.sum(compat0T * leftT_ref[0], axis=(0, 1), keepdims=True)
              + jnp.sum(compat1T * rightT_ref[0], axis=(0, 1), keepdims=True))

    aT = jnp.exp(compat0T) + jnp.exp(compat1T)

    z = jnp.dot(h, wr1_ref[...], preferred_element_type=f32) + br1_ref[0:1, :]
    z = 0.5 * z * (1.0 + lax.erf(z * f32(0.7071067811865476)))
    root_col = jnp.dot(z, wr2_ref[...], preferred_element_type=f32) + br2_ref[0, 0]
    gold_r = jnp.sum(root_col * r1h_ref[0], axis=(0, 1), keepdims=True)

    deg = jnp.sum(aT, axis=1, keepdims=True)
    jjd = lax.broadcasted_iota(jnp.int32, (n, n), 0)
    iid = lax.broadcasted_iota(jnp.int32, (n, n), 1)
    lapT = jnp.where(jjd == iid, deg - aT, -aT)
    lapT = jnp.where(iid == 0, jnp.exp(root_col), lapT)
    lapT_ref[0] = lapT
    gold_ref[0] = jnp.broadcast_to(gold_c + gold_r, gold_ref.shape[1:])


def _bdot(lhs, rhs, lc, rc):
    return lax.dot_general(lhs, rhs, (((lc,), (rc,)), ((0,), (0,))),
                           preferred_element_type=jnp.float32,
                           precision=lax.Precision.HIGHEST)


def _lu_kernel(lapT_ref, logabs_ref, sign_ref, mt_ref, qp_ref):
    f32 = jnp.float32
    b, n, _ = lapT_ref.shape
    w = _W
    lapT = lapT_ref[...]
    s = jnp.max(jnp.abs(lapT), axis=2, keepdims=True)
    mt_ref[...] = lapT / s
    scale_log = jnp.sum(jnp.log(s), axis=1)

    neg_inf = f32(-jnp.inf)
    il = lax.broadcasted_iota(jnp.int32, (b, 1, n), 2)
    isub = lax.broadcasted_iota(jnp.int32, (b, n, 1), 1)
    js = lax.broadcasted_iota(jnp.int32, (b, w, 1), 1)
    eye_w = jnp.where(
        lax.broadcasted_iota(jnp.int32, (b, w, w), 1)
        == lax.broadcasted_iota(jnp.int32, (b, w, w), 2), f32(1.0), f32(0.0))

    elim = jnp.zeros((b, 1, n), f32)
    signp = jnp.ones((b, 1, 1), f32)
    logabs = jnp.zeros((b, 1, 1), f32)
    pc = jnp.zeros((b, 1, n), f32)

    for q in range(0, n, w):
        trail = n - q - w

        def body(_, carry, q=q):
            k, elim, signp, logabs, pc = carry
            g = q + k
            colk = mt_ref[:, pl.ds(g, 1), :]
            am = jnp.where(elim > 0.5, neg_inf, jnp.abs(colk))
            kb = lax.bitcast_convert_type(am, jnp.int32)
            key = ((kb & jnp.int32(-512))
                   | jnp.where(colk < 0, jnp.int32(256), jnp.int32(0))
                   | (jnp.int32(n - 1) - il))
            keymax = jnp.max(key, axis=2, keepdims=True)
            piv = jnp.int32(n - 1) - (keymax & jnp.int32(255))
            pabs = lax.bitcast_convert_type(keymax & jnp.int32(-512), f32)
            pneg = (keymax & jnp.int32(256)) != 0
            psign = jnp.where(pneg, f32(-1.0), f32(1.0))
            is_p = il == piv
            rinv = psign * jnp.where(pabs == 0.0, f32(1.0), 1.0 / pabs)
            elim = elim + jnp.where(is_p, f32(1.0), f32(0.0))
            f = jnp.where(elim > 0.5, f32(0.0), colk) * rinv

            ptv = mt_ref[:, q:q + w, :]
            u = jnp.sum(jnp.where(is_p, ptv, f32(0.0)), axis=2, keepdims=True)
            mt_ref[:, q:q + w, :] = jnp.where(
                js == k, f, ptv - jnp.where(js > k, u * f, f32(0.0)))
            qp_ref[:, pl.ds(k, 1), :] = jnp.where(is_p, f32(1.0), f32(0.0))

            pc = jnp.where(il == g, piv.astype(f32), pc)
            signp = signp * jnp.where(pabs == 0.0, f32(0.0), psign)
            logabs = logabs + jnp.log(pabs)
            return jnp.int32(k + 1), elim, signp, logabs, pc

        init = (jnp.int32(0), elim, signp, logabs, pc)
        _, elim, signp, logabs, pc = lax.fori_loop(0, w, body, init)

        if trail > 0:
            fpan = mt_ref[:, q:q + w, :]
            qp = qp_ref[...]
            a12t = _bdot(mt_ref[:, q + w:, :], qp, 2, 2)
            nt = _bdot(fpan, qp, 2, 2)
            x = eye_w - nt
            at = eye_w + nt
            for _ in range(max(2, (w - 1).bit_length() - 1)):
                ax = _bdot(at, x, 2, 1)
                x = _bdot(x, 2.0 * eye_w - ax, 2, 1)
            u12t = _bdot(a12t, x, 2, 1)
            schur_t = _bdot(u12t, fpan, 2, 1)
            mt_ref[:, q + w:, :] = mt_ref[:, q + w:, :] - schur_t

    pr = lax.transpose(pc, (0, 2, 1))
    inv_cnt = jnp.sum(
        jnp.where(jnp.logical_and(isub < il, pr > pc), f32(1.0), f32(0.0)),
        axis=(1, 2), keepdims=True)
    inv_mod2 = inv_cnt - 2.0 * jnp.floor(inv_cnt * 0.5)
    parity = jnp.where(inv_mod2 > 0.5, f32(-1.0), f32(1.0))
    sign = signp * parity

    logabs_ref[...] = jnp.broadcast_to(logabs[:, 0, :] + scale_log,
                                       logabs_ref.shape)
    sign_ref[...] = jnp.broadcast_to(sign[:, 0, :], sign_ref.shape)


def kernel(h_cat, left_adj, right_adj, W_bilin, b_bilin, W_head, W_dep,
           W_r1, b_r1, W_r2, b_r2, roots):
    f32 = jnp.float32
    b, n, h = h_cat.shape
    roots1h = jax.nn.one_hot(roots, n, dtype=f32).reshape(b, n, 1)

    lapT, gold = pl.pallas_call(
        _compat_kernel,
        grid=(b,),
        in_specs=[
            pl.BlockSpec((1, n, h), lambda i: (i, _Z, _Z)),
            pl.BlockSpec((1, n, n), lambda i: (i, _Z, _Z)),
            pl.BlockSpec((1, n, n), lambda i: (i, _Z, _Z)),
            pl.BlockSpec((2, h, h), lambda i: (_Z, _Z, _Z)),
            pl.BlockSpec((1, 2), lambda i: (_Z, _Z)),
            pl.BlockSpec((2, h), lambda i: (_Z, _Z)),
            pl.BlockSpec((2, h), lambda i: (_Z, _Z)),
            pl.BlockSpec((h, h), lambda i: (_Z, _Z)),
            pl.BlockSpec((1, h), lambda i: (_Z, _Z)),
            pl.BlockSpec((h, 1), lambda i: (_Z, _Z)),
            pl.BlockSpec((1, 1), lambda i: (_Z, _Z)),
            pl.BlockSpec((1, n, 1), lambda i: (i, _Z, _Z)),
        ],
        out_specs=[
            pl.BlockSpec((1, n, n), lambda i: (i, _Z, _Z)),
            pl.BlockSpec((1, 1, 128), lambda i: (i, _Z, _Z)),
        ],
        out_shape=[
            jax.ShapeDtypeStruct((b, n, n), f32),
            jax.ShapeDtypeStruct((b, 1, 128), f32),
        ],
    )(h_cat, jnp.swapaxes(left_adj, 1, 2), jnp.swapaxes(right_adj, 1, 2),
      W_bilin, b_bilin.reshape(1, 2).astype(f32), W_head, W_dep,
      W_r1, b_r1.reshape(1, h).astype(f32), W_r2,
      b_r2.reshape(1, 1).astype(f32), roots1h)

    logabs, sign = pl.pallas_call(
        _lu_kernel,
        out_shape=[
            jax.ShapeDtypeStruct((b, 128), f32),
            jax.ShapeDtypeStruct((b, 128), f32),
        ],
        scratch_shapes=[pltpu.VMEM((b, n, n), f32),
                        pltpu.VMEM((b, _W, n), f32)],
    )(lapT)

    gold_v = gold[:, 0, 0].astype(jnp.float64)
    la = logabs[:, 0].astype(jnp.float64)
    sg = sign[:, 0]
    logdet = jnp.where(sg > 0, la, jnp.nan)
    valid = jnp.logical_and(~jnp.isnan(gold_v), ~jnp.isnan(logdet)).astype(jnp.float64)
    mask = (gold_v <= logdet * valid).astype(jnp.float64)
    loss = (logdet - gold_v) * mask
    loss = jnp.where(jnp.isnan(loss), 0.0, loss)
    return _ALPHA * jnp.sum(loss) / b


# speedup vs baseline: 1.2923x; 1.0036x over previous
import jax
import jax.numpy as jnp
import numpy as np
from jax import lax
from jax.experimental import pallas as pl
from jax.experimental.pallas import tpu as pltpu

_ALPHA = 0.25
_Z = np.int32(0)
_W = 32


def _compat_kernel(h_ref, leftT_ref, rightT_ref, wb_ref, bb_ref, wh_ref,
                   wd_ref, wr1_ref, br1_ref, wr2_ref, br2_ref, r1h_ref,
                   lapT_ref, gold_ref):
    f32 = jnp.float32
    h = h_ref[0]
    n = h.shape[0]

    dn_et = (((1,), (1,)), ((), ()))
    tmp0 = jnp.dot(h, wb_ref[0], preferred_element_type=f32)
    tmp1 = jnp.dot(h, wb_ref[1], preferred_element_type=f32)
    bilin0T = lax.dot_general(h, tmp0, dn_et, preferred_element_type=f32)
    bilin1T = lax.dot_general(h, tmp1, dn_et, preferred_element_type=f32)

    head0 = lax.dot_general(wh_ref[0:1, :], h, dn_et, preferred_element_type=f32)
    head1 = lax.dot_general(wh_ref[1:2, :], h, dn_et, preferred_element_type=f32)
    dep0 = lax.dot_general(h, wd_ref[0:1, :], dn_et, preferred_element_type=f32)
    dep1 = lax.dot_general(h, wd_ref[1:2, :], dn_et, preferred_element_type=f32)

    compat0T = bilin0T + head0 + dep0 + bb_ref[0, 0]
    compat1T = bilin1T + head1 + dep1 + bb_ref[0, 1]

    gold_c = (jnp.sum(compat0T * leftT_ref[0], axis=(0, 1), keepdims=True)
              + jnp.sum(compat1T * rightT_ref[0], axis=(0, 1), keepdims=True))

    aT = jnp.exp(compat0T) + jnp.exp(compat1T)

    z = jnp.dot(h, wr1_ref[...], preferred_element_type=f32) + br1_ref[0:1, :]
    z = 0.5 * z * (1.0 + lax.erf(z * f32(0.7071067811865476)))
    root_col = jnp.dot(z, wr2_ref[...], preferred_element_type=f32) + br2_ref[0, 0]
    gold_r = jnp.sum(root_col * r1h_ref[0], axis=(0, 1), keepdims=True)

    deg = jnp.sum(aT, axis=1, keepdims=True)
    jjd = lax.broadcasted_iota(jnp.int32, (n, n), 0)
    iid = lax.broadcasted_iota(jnp.int32, (n, n), 1)
    lapT = jnp.where(jjd == iid, deg - aT, -aT)
    lapT = jnp.where(iid == 0, jnp.exp(root_col), lapT)
    lapT_ref[0] = lapT
    gold_ref[0] = jnp.broadcast_to(gold_c + gold_r, gold_ref.shape[1:])


def _bdot(lhs, rhs, lc, rc):
    return lax.dot_general(lhs, rhs, (((lc,), (rc,)), ((0,), (0,))),
                           preferred_element_type=jnp.float32,
                           precision=lax.Precision.HIGHEST)


def _lu_kernel(lapT_ref, logabs_ref, sign_ref, mt_ref, qp_ref):
    f32 = jnp.float32
    b, n, _ = lapT_ref.shape
    w = _W
    lapT = lapT_ref[...]
    s = jnp.max(jnp.abs(lapT), axis=2, keepdims=True)
    mt_ref[...] = lapT / s
    scale_log = jnp.sum(jnp.log(s), axis=1)

    neg_inf = f32(-jnp.inf)
    il = lax.broadcasted_iota(jnp.int32, (b, 1, n), 2)
    isub = lax.broadcasted_iota(jnp.int32, (b, n, 1), 1)
    js = lax.broadcasted_iota(jnp.int32, (b, w, 1), 1)
    eye_w = jnp.where(
        lax.broadcasted_iota(jnp.int32, (b, w, w), 1)
        == lax.broadcasted_iota(jnp.int32, (b, w, w), 2), f32(1.0), f32(0.0))

    elim = jnp.zeros((b, 1, n), f32)
    signp = jnp.ones((b, 1, 1), f32)
    logabs = jnp.zeros((b, 1, 1), f32)
    pc = jnp.zeros((b, 1, n), f32)

    for q in range(0, n, w):
        trail = n - q - w

        def body(_, carry, q=q):
            k, elim, signp, logabs, pc = carry
            g = q + k
            colk = mt_ref[:, pl.ds(g, 1), :]
            am = jnp.where(elim > 0.5, neg_inf, jnp.abs(colk))
            kb = lax.bitcast_convert_type(am, jnp.int32)
            key = ((kb & jnp.int32(-512))
                   | jnp.where(colk < 0, jnp.int32(256), jnp.int32(0))
                   | (jnp.int32(n - 1) - il))
            keymax = jnp.max(key, axis=2, keepdims=True)
            piv = jnp.int32(n - 1) - (keymax & jnp.int32(255))
            pabs = lax.bitcast_convert_type(keymax & jnp.int32(-512), f32)
            pneg = (keymax & jnp.int32(256)) != 0
            psign = jnp.where(pneg, f32(-1.0), f32(1.0))
            is_p = il == piv
            rinv = psign * jnp.where(pabs == 0.0, f32(1.0), 1.0 / pabs)
            elim = elim + jnp.where(is_p, f32(1.0), f32(0.0))
            f = jnp.where(elim > 0.5, f32(0.0), colk) * rinv

            ptv = mt_ref[:, q:q + w, :]
            u = jnp.sum(jnp.where(is_p, ptv, f32(0.0)), axis=2, keepdims=True)
            mt_ref[:, q:q + w, :] = jnp.where(
                js == k, f, ptv - jnp.where(js > k, u * f, f32(0.0)))
            qp_ref[:, pl.ds(k, 1), :] = jnp.where(is_p, f32(1.0), f32(0.0))

            pc = jnp.where(il == g, piv.astype(f32), pc)
            signp = signp * jnp.where(pabs == 0.0, f32(0.0), psign)
            logabs = logabs + jnp.log(pabs)
            return jnp.int32(k + 1), elim, signp, logabs, pc

        init = (jnp.int32(0), elim, signp, logabs, pc)
        _, elim, signp, logabs, pc = lax.fori_loop(0, w, body, init)

        if trail > 0:
            fpan = mt_ref[:, q:q + w, :]
            qp = qp_ref[...]
            a12t = _bdot(mt_ref[:, q + w:, :], qp, 2, 2)
            nt = _bdot(fpan, qp, 2, 2)
            x = eye_w - nt
            at = eye_w + nt
            for _ in range(max(2, (w - 1).bit_length() - 1)):
                ax = _bdot(at, x, 2, 1)
                x = _bdot(x, 2.0 * eye_w - ax, 2, 1)
            u12t = _bdot(a12t, x, 2, 1)
            schur_t = _bdot(u12t, fpan, 2, 1)
            mt_ref[:, q + w:, :] = mt_ref[:, q + w:, :] - schur_t

    pr = lax.transpose(pc, (0, 2, 1))
    inv_cnt = jnp.sum(
        jnp.where(jnp.logical_and(isub < il, pr > pc), f32(1.0), f32(0.0)),
        axis=(1, 2), keepdims=True)
    inv_mod2 = inv_cnt - 2.0 * jnp.floor(inv_cnt * 0.5)
    parity = jnp.where(inv_mod2 > 0.5, f32(-1.0), f32(1.0))
    sign = signp * parity

    logabs_ref[...] = jnp.broadcast_to(logabs[:, 0, :] + scale_log,
                                       logabs_ref.shape)
    sign_ref[...] = jnp.broadcast_to(sign[:, 0, :], sign_ref.shape)


def kernel(h_cat, left_adj, right_adj, W_bilin, b_bilin, W_head, W_dep,
           W_r1, b_r1, W_r2, b_r2, roots):
    f32 = jnp.float32
    b, n, h = h_cat.shape
    roots1h = jax.nn.one_hot(roots, n, dtype=f32).reshape(b, n, 1)

    lapT, gold = pl.pallas_call(
        _compat_kernel,
        grid=(b,),
        in_specs=[
            pl.BlockSpec((1, n, h), lambda i: (i, _Z, _Z)),
            pl.BlockSpec((1, n, n), lambda i: (i, _Z, _Z)),
            pl.BlockSpec((1, n, n), lambda i: (i, _Z, _Z)),
            pl.BlockSpec((2, h, h), lambda i: (_Z, _Z, _Z)),
            pl.BlockSpec((1, 2), lambda i: (_Z, _Z)),
            pl.BlockSpec((2, h), lambda i: (_Z, _Z)),
            pl.BlockSpec((2, h), lambda i: (_Z, _Z)),
            pl.BlockSpec((h, h), lambda i: (_Z, _Z)),
            pl.BlockSpec((1, h), lambda i: (_Z, _Z)),
            pl.BlockSpec((h, 1), lambda i: (_Z, _Z)),
            pl.BlockSpec((1, 1), lambda i: (_Z, _Z)),
            pl.BlockSpec((1, n, 1), lambda i: (i, _Z, _Z)),
        ],
        out_specs=[
            pl.BlockSpec((1, n, n), lambda i: (i, _Z, _Z)),
            pl.BlockSpec((1, 1, 128), lambda i: (i, _Z, _Z)),
        ],
        out_shape=[
            jax.ShapeDtypeStruct((b, n, n), f32),
            jax.ShapeDtypeStruct((b, 1, 128), f32),
        ],
    )(h_cat, jnp.swapaxes(left_adj, 1, 2), jnp.swapaxes(right_adj, 1, 2),
      W_bilin, b_bilin.reshape(1, 2).astype(f32), W_head, W_dep,
      W_r1, b_r1.reshape(1, h).astype(f32), W_r2,
      b_r2.reshape(1, 1).astype(f32), roots1h)

    logabs, sign = pl.pallas_call(
        _lu_kernel,
        out_shape=[
            jax.ShapeDtypeStruct((b, 128), f32),
            jax.ShapeDtypeStruct((b, 128), f32),
        ],
        scratch_shapes=[pltpu.VMEM((b, n, n), f32),
                        pltpu.VMEM((b, _W, n), f32)],
    )(lapT)

    gold_v = gold[:, 0, 0].astype(jnp.float64)
    la = logabs[:, 0].astype(jnp.float64)
    sg = sign[:, 0]
    logdet = jnp.where(sg > 0, la, jnp.nan)
    valid = jnp.logical_and(~jnp.isnan(gold_v), ~jnp.isnan(logdet)).astype(jnp.float64)
    mask = (gold_v <= logdet * valid).astype(jnp.float64)
    loss = (logdet - gold_v) * mask
    loss = jnp.where(jnp.isnan(loss), 0.0, loss)
    return _ALPHA * jnp.sum(loss) / b
